# baseline (speedup 1.0000x reference)
import jax
import jax.numpy as jnp
from jax.experimental import pallas as pl
from jax.experimental.pallas import tpu as pltpu

N_EDGES = 20000
D = 256
BLK_IN = 5000
NB_IN = N_EDGES // BLK_IN
BLK_OUT = 2000
NB_OUT = N_EDGES // BLK_OUT


def _mp_kernel(x_ref, w3t_ref, w3b_ref, b3_ref, out_ref,
               z_scr, pooled_scr, cvec_scr):
    s = pl.program_id(0)

    @pl.when(s < NB_IN)
    def _phase_stream():
        z_scr[pl.ds(s * BLK_IN, BLK_IN), :] = jnp.dot(
            x_ref[...], w3t_ref[...], preferred_element_type=jnp.float32)
        blk_max = jnp.max(x_ref[...], axis=0, keepdims=True)

        @pl.when(s == 0)
        def _():
            pooled_scr[...] = blk_max

        @pl.when(s > 0)
        def _():
            pooled_scr[...] = jnp.maximum(pooled_scr[...], blk_max)

        @pl.when(s == NB_IN - 1)
        def _():
            cvec_scr[...] = (
                jnp.dot(pooled_scr[...], w3b_ref[...],
                        preferred_element_type=jnp.float32)
                + b3_ref[...]
            )

    @pl.when(s >= NB_IN)
    def _phase_emit():
        j = s - NB_IN
        out_ref[...] = jnp.maximum(
            z_scr[pl.ds(j * BLK_OUT, BLK_OUT), :] + cvec_scr[...], 0.0)


def kernel(edge_pred, edge_corner, all_corners, edge_x, image_x, W3, b3,
           interpret=False):
    del edge_pred, edge_corner, all_corners, image_x
    w3t = W3[:D, :]
    w3b = W3[D:, :]
    b3_2d = b3.reshape(1, D)

    out = pl.pallas_call(
        _mp_kernel,
        grid=(NB_IN + NB_OUT,),
        in_specs=[
            pl.BlockSpec((BLK_IN, D),
                         lambda s: (jnp.minimum(s, NB_IN - 1), 0)),
            pl.BlockSpec((D, D), lambda s: (0, 0)),
            pl.BlockSpec((D, D), lambda s: (0, 0)),
            pl.BlockSpec((1, D), lambda s: (0, 0)),
        ],
        out_specs=pl.BlockSpec((BLK_OUT, D),
                               lambda s: (jnp.maximum(s - NB_IN, 0), 0)),
        out_shape=jax.ShapeDtypeStruct((N_EDGES, D), jnp.float32),
        scratch_shapes=[
            pltpu.VMEM((N_EDGES, D), jnp.float32),
            pltpu.VMEM((1, D), jnp.float32),
            pltpu.VMEM((1, D), jnp.float32),
        ],
        interpret=interpret,
    )(edge_x, w3t, w3b, b3_2d)
    return out


# speedup vs baseline: 1.0483x; 1.0483x over previous
import jax
import jax.numpy as jnp
from jax.experimental import pallas as pl
from jax.experimental.pallas import tpu as pltpu

N_EDGES = 20000
D = 256
BLK_IN = 5000
NB_IN = N_EDGES // BLK_IN
BLK_OUT = 4000
NB_OUT = N_EDGES // BLK_OUT


def _mp_kernel(x_ref, w3t_ref, w3b_ref, b3_ref, out_ref,
               z_scr, pooled_scr, cvec_scr):
    s = pl.program_id(0)

    @pl.when(s < NB_IN)
    def _phase_stream():
        z_scr[pl.ds(s * BLK_IN, BLK_IN), :] = jnp.dot(
            x_ref[...], w3t_ref[...], preferred_element_type=jnp.float32)
        blk_max = jnp.max(x_ref[...], axis=0, keepdims=True)

        @pl.when(s == 0)
        def _():
            pooled_scr[...] = blk_max

        @pl.when(s > 0)
        def _():
            pooled_scr[...] = jnp.maximum(pooled_scr[...], blk_max)

        @pl.when(s == NB_IN - 1)
        def _():
            cvec_scr[...] = (
                jnp.dot(pooled_scr[...], w3b_ref[...],
                        preferred_element_type=jnp.float32)
                + b3_ref[...]
            )

    @pl.when(s >= NB_IN)
    def _phase_emit():
        j = s - NB_IN
        out_ref[...] = jnp.maximum(
            z_scr[pl.ds(j * BLK_OUT, BLK_OUT), :] + cvec_scr[...], 0.0)


def kernel(edge_pred, edge_corner, all_corners, edge_x, image_x, W3, b3,
           interpret=False):
    del edge_pred, edge_corner, all_corners, image_x
    w3t = W3[:D, :]
    w3b = W3[D:, :]
    b3_2d = b3.reshape(1, D)

    out = pl.pallas_call(
        _mp_kernel,
        grid=(NB_IN + NB_OUT,),
        in_specs=[
            pl.BlockSpec((BLK_IN, D),
                         lambda s: (jnp.minimum(s, NB_IN - 1), 0)),
            pl.BlockSpec((D, D), lambda s: (0, 0)),
            pl.BlockSpec((D, D), lambda s: (0, 0)),
            pl.BlockSpec((1, D), lambda s: (0, 0)),
        ],
        out_specs=pl.BlockSpec((BLK_OUT, D),
                               lambda s: (jnp.maximum(s - NB_IN, 0), 0)),
        out_shape=jax.ShapeDtypeStruct((N_EDGES, D), jnp.float32),
        scratch_shapes=[
            pltpu.VMEM((N_EDGES, D), jnp.float32),
            pltpu.VMEM((1, D), jnp.float32),
            pltpu.VMEM((1, D), jnp.float32),
        ],
        interpret=interpret,
    )(edge_x, w3t, w3b, b3_2d)
    return out
